# Initial kernel scaffold; baseline (speedup 1.0000x reference)
import jax
import jax.numpy as jnp
from jax.experimental import pallas as pl


def kernel(x, edge_index, edge_attr, batch_vec, W_in, W_edge, W1, b1, W2, b2, eps, W_pred, b_pred):
    raise NotImplementedError("write your pallas kernel here")


# speedup vs baseline: 2.0990x; 2.0990x over previous
import functools

import jax
import jax.numpy as jnp
from jax import lax
from jax.experimental import pallas as pl
from jax.experimental.pallas import tpu as pltpu
from jax.experimental.pallas import tpu_sc as plsc

N_NODES = 10000
N_EDGES = 320000
IN_DIM = 128
EMB = 128
EDGE_DIM = 16
N_LAYERS = 3
N_GRAPHS = 64
N_CLASS = 10

SC_CORES = 2
SC_SUBCORES = 16
HALF = EMB // SC_CORES
ROWS_PER_TILE = N_NODES // SC_SUBCORES
EDGES_PER_TILE = N_EDGES // SC_SUBCORES
CHUNK = 80
N_CHUNKS = EDGES_PER_TILE // CHUNK


def _encode_body(x_ref, w_ref, o_ref):
    o_ref[...] = jnp.dot(x_ref[...], w_ref[...],
                         preferred_element_type=jnp.float32)


def _encode(x, w_in, interpret=False):
    bm = 2000
    return pl.pallas_call(
        _encode_body,
        grid=(N_NODES // bm,),
        in_specs=[
            pl.BlockSpec((bm, IN_DIM), lambda i: (i, 0)),
            pl.BlockSpec((IN_DIM, EMB), lambda i: (0, 0)),
        ],
        out_specs=pl.BlockSpec((bm, EMB), lambda i: (i, 0)),
        out_shape=jax.ShapeDtypeStruct((N_NODES, EMB), jnp.float32),
        interpret=interpret,
    )(x, w_in)


def _edge_emb_body(ea_ref, w_ref, o0_ref, o1_ref, o2_ref):
    ea = ea_ref[...]
    o0_ref[...] = jnp.dot(ea, w_ref[0], preferred_element_type=jnp.float32)
    o1_ref[...] = jnp.dot(ea, w_ref[1], preferred_element_type=jnp.float32)
    o2_ref[...] = jnp.dot(ea, w_ref[2], preferred_element_type=jnp.float32)


def _edge_emb_all(edge_attr, w_edge, interpret=False):
    bm = 4000
    out = jax.ShapeDtypeStruct((N_EDGES, EMB), jnp.float32)
    return pl.pallas_call(
        _edge_emb_body,
        grid=(N_EDGES // bm,),
        in_specs=[
            pl.BlockSpec((bm, EDGE_DIM), lambda i: (i, 0)),
            pl.BlockSpec((N_LAYERS, EDGE_DIM, EMB), lambda i: (0, 0, 0)),
        ],
        out_specs=[pl.BlockSpec((bm, EMB), lambda i: (i, 0))] * 3,
        out_shape=[out, out, out],
        interpret=interpret,
    )(edge_attr, w_edge)


def _mlp_body(h_ref, agg_ref, eps_ref, w1_ref, b1_ref, w2_ref, b2_ref, o_ref,
              *, relu_out):
    z = (1.0 + eps_ref[0, 0]) * h_ref[...] + agg_ref[...]
    a = jnp.maximum(jnp.dot(z, w1_ref[...],
                            preferred_element_type=jnp.float32)
                    + b1_ref[...], 0.0)
    o = jnp.dot(a, w2_ref[...], preferred_element_type=jnp.float32) + b2_ref[...]
    if relu_out:
        o = jnp.maximum(o, 0.0)
    o_ref[...] = o


def _mlp(h, agg, eps_l, w1, b1, w2, b2, relu_out, interpret=False):
    bm = 2000
    return pl.pallas_call(
        functools.partial(_mlp_body, relu_out=relu_out),
        grid=(N_NODES // bm,),
        in_specs=[
            pl.BlockSpec((bm, EMB), lambda i: (i, 0)),
            pl.BlockSpec((bm, EMB), lambda i: (i, 0)),
            pl.BlockSpec((1, 1), lambda i: (0, 0)),
            pl.BlockSpec((EMB, 2 * EMB), lambda i: (0, 0)),
            pl.BlockSpec((1, 2 * EMB), lambda i: (0, 0)),
            pl.BlockSpec((2 * EMB, EMB), lambda i: (0, 0)),
            pl.BlockSpec((1, EMB), lambda i: (0, 0)),
        ],
        out_specs=pl.BlockSpec((bm, EMB), lambda i: (i, 0)),
        out_shape=jax.ShapeDtypeStruct((N_NODES, EMB), jnp.float32),
        interpret=interpret,
    )(h, agg, eps_l.reshape(1, 1), w1, b1.reshape(1, -1), w2,
      b2.reshape(1, -1))


def _pool_head_body(h_ref, bv_ref, wp_ref, bp_ref, pred_ref, hg_ref):
    bv = bv_ref[...]
    gids = lax.broadcasted_iota(jnp.int32, (N_GRAPHS, N_NODES), 0)
    oh = (gids == bv).astype(jnp.float32)
    sums = jnp.dot(oh, h_ref[...], preferred_element_type=jnp.float32)
    counts = jnp.sum(oh, axis=1, keepdims=True)
    hg = sums / jnp.maximum(counts, 1.0)
    hg_ref[...] = hg
    pred_ref[...] = jnp.dot(hg, wp_ref[...],
                            preferred_element_type=jnp.float32) + bp_ref[...]


def _pool_head(h, batch_vec, w_pred, b_pred, interpret=False):
    return pl.pallas_call(
        _pool_head_body,
        in_specs=[
            pl.BlockSpec((N_NODES, EMB), lambda: (0, 0)),
            pl.BlockSpec((1, N_NODES), lambda: (0, 0)),
            pl.BlockSpec((EMB, N_CLASS), lambda: (0, 0)),
            pl.BlockSpec((1, N_CLASS), lambda: (0, 0)),
        ],
        out_specs=[
            pl.BlockSpec((N_GRAPHS, N_CLASS), lambda: (0, 0)),
            pl.BlockSpec((N_GRAPHS, EMB), lambda: (0, 0)),
        ],
        out_shape=[
            jax.ShapeDtypeStruct((N_GRAPHS, N_CLASS), jnp.float32),
            jax.ShapeDtypeStruct((N_GRAPHS, EMB), jnp.float32),
        ],
        interpret=interpret,
    )(h, batch_vec.reshape(1, -1), w_pred, b_pred.reshape(1, -1))


def _gin_edge_body(h_hbm, e_hbm, src_hbm, dst_hbm, zeros_hbm, out_hbm,
                   h_sh, agg_sh, src_v, dst_v, e_v, g_v, sem):
    cid = lax.axis_index("c")
    sid = lax.axis_index("s")
    col0 = cid * HALF
    r0 = sid * ROWS_PER_TILE

    pltpu.sync_copy(h_hbm.at[pl.ds(r0, ROWS_PER_TILE), pl.ds(col0, HALF)],
                    h_sh.at[pl.ds(r0, ROWS_PER_TILE), :])
    pltpu.sync_copy(zeros_hbm, agg_sh.at[pl.ds(r0, ROWS_PER_TILE), :])
    plsc.subcore_barrier()

    def chunk_body(i, carry):
        base = sid * EDGES_PER_TILE + i * CHUNK
        pltpu.sync_copy(src_hbm.at[pl.ds(base, CHUNK)], src_v)
        pltpu.sync_copy(dst_hbm.at[pl.ds(base, CHUNK)], dst_v)
        pltpu.sync_copy(e_hbm.at[pl.ds(base, CHUNK), pl.ds(col0, HALF)], e_v)
        pltpu.async_copy(h_sh.at[src_v], g_v, sem).wait()

        def edge_body(j, c2):
            for v in range(HALF // 16):
                sl = pl.ds(v * 16, 16)
                e_v[j, sl] = jnp.maximum(g_v[j, sl] + e_v[j, sl], 0.0)
            return c2

        lax.fori_loop(0, CHUNK, edge_body, 0)
        pltpu.sync_copy(e_v, agg_sh.at[dst_v], add=True)
        return carry

    lax.fori_loop(0, N_CHUNKS, chunk_body, 0)
    plsc.subcore_barrier()

    pltpu.sync_copy(agg_sh.at[pl.ds(r0, ROWS_PER_TILE), :],
                    out_hbm.at[pl.ds(r0, ROWS_PER_TILE), pl.ds(col0, HALF)])


def _gin_edge(h, e_emb, src, dst, zeros, interpret=False):
    mesh = plsc.VectorSubcoreMesh(core_axis_name="c", subcore_axis_name="s",
                                  num_cores=SC_CORES,
                                  num_subcores=SC_SUBCORES)
    f = pl.kernel(
        _gin_edge_body,
        out_type=jax.ShapeDtypeStruct((N_NODES, EMB), jnp.float32),
        mesh=mesh,
        scratch_types=[
            pltpu.VMEM_SHARED((N_NODES, HALF), jnp.float32),
            pltpu.VMEM_SHARED((N_NODES, HALF), jnp.float32),
            pltpu.VMEM((CHUNK,), jnp.int32),
            pltpu.VMEM((CHUNK,), jnp.int32),
            pltpu.VMEM((CHUNK, HALF), jnp.float32),
            pltpu.VMEM((CHUNK, HALF), jnp.float32),
            pltpu.SemaphoreType.DMA,
        ],
        compiler_params=pltpu.CompilerParams(use_tc_tiling_on_sc=False),
        interpret=interpret,
    )
    return f(h, e_emb, src, dst, zeros)


def kernel(x, edge_index, edge_attr, batch_vec, W_in, W_edge, W1, b1, W2, b2,
           eps, W_pred, b_pred):
    src = edge_index[0].astype(jnp.int32)
    dst = edge_index[1].astype(jnp.int32)
    zeros = jnp.zeros((ROWS_PER_TILE, HALF), jnp.float32)

    h = _encode(x, W_in)
    e_embs = _edge_emb_all(edge_attr, W_edge)
    for l in range(N_LAYERS):
        agg = _gin_edge(h, e_embs[l], src, dst, zeros)
        h = _mlp(h, agg, eps[l], W1[l], b1[l], W2[l], b2[l],
                 relu_out=(l < N_LAYERS - 1))
    pred, h_graph = _pool_head(h, batch_vec.astype(jnp.int32), W_pred, b_pred)
    return (pred, h_graph)


# speedup vs baseline: 2.5223x; 1.2016x over previous
import functools

import jax
import jax.numpy as jnp
from jax import lax
from jax.experimental import pallas as pl
from jax.experimental.pallas import tpu as pltpu
from jax.experimental.pallas import tpu_sc as plsc

N_NODES = 10000
N_EDGES = 320000
IN_DIM = 128
EMB = 128
EDGE_DIM = 16
N_LAYERS = 3
N_GRAPHS = 64
N_CLASS = 10

SC_CORES = 2
SC_SUBCORES = 16
HALF = EMB // SC_CORES
ROWS_PER_TILE = N_NODES // SC_SUBCORES
EDGES_PER_TILE = N_EDGES // SC_SUBCORES
SUB = 100
KSUB = 2
SUPER = SUB * KSUB
N_SUPER = EDGES_PER_TILE // SUPER
IDX_ROWS = N_EDGES // SUB


def _encode_body(x_ref, w_ref, o_ref):
    o_ref[...] = jnp.dot(x_ref[...], w_ref[...],
                         preferred_element_type=jnp.float32)


def _encode(x, w_in, interpret=False):
    bm = 2000
    return pl.pallas_call(
        _encode_body,
        grid=(N_NODES // bm,),
        in_specs=[
            pl.BlockSpec((bm, IN_DIM), lambda i: (i, 0)),
            pl.BlockSpec((IN_DIM, EMB), lambda i: (0, 0)),
        ],
        out_specs=pl.BlockSpec((bm, EMB), lambda i: (i, 0)),
        out_shape=jax.ShapeDtypeStruct((N_NODES, EMB), jnp.float32),
        interpret=interpret,
    )(x, w_in)


def _edge_emb_body(ea_ref, w_ref, o0_ref, o1_ref, o2_ref):
    ea = ea_ref[...]
    o0_ref[...] = jnp.dot(ea, w_ref[0], preferred_element_type=jnp.float32)
    o1_ref[...] = jnp.dot(ea, w_ref[1], preferred_element_type=jnp.float32)
    o2_ref[...] = jnp.dot(ea, w_ref[2], preferred_element_type=jnp.float32)


def _edge_emb_all(edge_attr, w_edge, interpret=False):
    bm = 4000
    out = jax.ShapeDtypeStruct((N_EDGES, EMB), jnp.float32)
    return pl.pallas_call(
        _edge_emb_body,
        grid=(N_EDGES // bm,),
        in_specs=[
            pl.BlockSpec((bm, EDGE_DIM), lambda i: (i, 0)),
            pl.BlockSpec((N_LAYERS, EDGE_DIM, EMB), lambda i: (0, 0, 0)),
        ],
        out_specs=[pl.BlockSpec((bm, EMB), lambda i: (i, 0))] * 3,
        out_shape=[out, out, out],
        interpret=interpret,
    )(edge_attr, w_edge)


def _mlp_body(h_ref, agg_ref, eps_ref, w1_ref, b1_ref, w2_ref, b2_ref, o_ref,
              *, relu_out):
    z = (1.0 + eps_ref[0, 0]) * h_ref[...] + agg_ref[...]
    a = jnp.maximum(jnp.dot(z, w1_ref[...],
                            preferred_element_type=jnp.float32)
                    + b1_ref[...], 0.0)
    o = jnp.dot(a, w2_ref[...], preferred_element_type=jnp.float32) + b2_ref[...]
    if relu_out:
        o = jnp.maximum(o, 0.0)
    o_ref[...] = o


def _mlp(h, agg, eps_l, w1, b1, w2, b2, relu_out, interpret=False):
    bm = 2000
    return pl.pallas_call(
        functools.partial(_mlp_body, relu_out=relu_out),
        grid=(N_NODES // bm,),
        in_specs=[
            pl.BlockSpec((bm, EMB), lambda i: (i, 0)),
            pl.BlockSpec((bm, EMB), lambda i: (i, 0)),
            pl.BlockSpec((1, 1), lambda i: (0, 0)),
            pl.BlockSpec((EMB, 2 * EMB), lambda i: (0, 0)),
            pl.BlockSpec((1, 2 * EMB), lambda i: (0, 0)),
            pl.BlockSpec((2 * EMB, EMB), lambda i: (0, 0)),
            pl.BlockSpec((1, EMB), lambda i: (0, 0)),
        ],
        out_specs=pl.BlockSpec((bm, EMB), lambda i: (i, 0)),
        out_shape=jax.ShapeDtypeStruct((N_NODES, EMB), jnp.float32),
        interpret=interpret,
    )(h, agg, eps_l.reshape(1, 1), w1, b1.reshape(1, -1), w2,
      b2.reshape(1, -1))


def _pool_head_body(h_ref, bv_ref, wp_ref, bp_ref, pred_ref, hg_ref):
    bv = bv_ref[...]
    gids = lax.broadcasted_iota(jnp.int32, (N_GRAPHS, N_NODES), 0)
    oh = (gids == bv).astype(jnp.float32)
    sums = jnp.dot(oh, h_ref[...], preferred_element_type=jnp.float32)
    counts = jnp.sum(oh, axis=1, keepdims=True)
    hg = sums / jnp.maximum(counts, 1.0)
    hg_ref[...] = hg
    pred_ref[...] = jnp.dot(hg, wp_ref[...],
                            preferred_element_type=jnp.float32) + bp_ref[...]


def _pool_head(h, batch_vec, w_pred, b_pred, interpret=False):
    return pl.pallas_call(
        _pool_head_body,
        in_specs=[
            pl.BlockSpec((N_NODES, EMB), lambda: (0, 0)),
            pl.BlockSpec((1, N_NODES), lambda: (0, 0)),
            pl.BlockSpec((EMB, N_CLASS), lambda: (0, 0)),
            pl.BlockSpec((1, N_CLASS), lambda: (0, 0)),
        ],
        out_specs=[
            pl.BlockSpec((N_GRAPHS, N_CLASS), lambda: (0, 0)),
            pl.BlockSpec((N_GRAPHS, EMB), lambda: (0, 0)),
        ],
        out_shape=[
            jax.ShapeDtypeStruct((N_GRAPHS, N_CLASS), jnp.float32),
            jax.ShapeDtypeStruct((N_GRAPHS, EMB), jnp.float32),
        ],
        interpret=interpret,
    )(h, batch_vec.reshape(1, -1), w_pred, b_pred.reshape(1, -1))


def _gin_edge_body(h_hbm, e_hbm, src_hbm, dst_hbm, zeros_hbm, out_hbm,
                   h_sh, agg_sh,
                   src0, dst0, e0, src1, dst1, e1, ga, gb,
                   ldsem0, ldsem1, gsa, gsb, sca, scb):
    cid = lax.axis_index("c")
    sid = lax.axis_index("s")
    col0 = cid * HALF
    r0 = sid * ROWS_PER_TILE
    row_base = sid * (EDGES_PER_TILE // SUB)
    e_base = sid * EDGES_PER_TILE

    def loads(c, srcb, dstb, eb, sem):
        rb = row_base + c * KSUB
        eo = e_base + c * SUPER
        pltpu.async_copy(src_hbm.at[pl.ds(rb, KSUB), :], srcb, sem)
        pltpu.async_copy(dst_hbm.at[pl.ds(rb, KSUB), :], dstb, sem)
        pltpu.async_copy(e_hbm.at[pl.ds(eo, SUPER), pl.ds(col0, HALF)], eb,
                         sem)

    def wait_loads(srcb, dstb, eb, sem):
        pltpu.make_async_copy(src_hbm.at[pl.ds(row_base, KSUB), :], srcb,
                              sem).wait()
        pltpu.make_async_copy(dst_hbm.at[pl.ds(row_base, KSUB), :], dstb,
                              sem).wait()
        pltpu.make_async_copy(
            e_hbm.at[pl.ds(e_base, SUPER), pl.ds(col0, HALF)], eb, sem).wait()

    def compute(eb, k, g):
        def edge_body(j, c2):
            row = k * SUB + j
            for v in range(HALF // 16):
                sl = pl.ds(v * 16, 16)
                g[j, sl] = jnp.maximum(g[j, sl] + eb[row, sl], 0.0)
            return c2

        lax.fori_loop(0, SUB, edge_body, 0, unroll=2)

    def process(srcb, dstb, eb):
        pltpu.async_copy(h_sh.at[srcb.at[0]], ga, gsa)
        pltpu.async_copy(h_sh.at[srcb.at[1]], gb, gsb)
        pltpu.make_async_copy(h_sh.at[srcb.at[0]], ga, gsa).wait()
        compute(eb, 0, ga)
        pltpu.async_copy(ga, agg_sh.at[dstb.at[0]], sca, add=True)
        pltpu.make_async_copy(h_sh.at[srcb.at[1]], gb, gsb).wait()
        compute(eb, 1, gb)
        pltpu.async_copy(gb, agg_sh.at[dstb.at[1]], scb, add=True)
        pltpu.make_async_copy(ga, agg_sh.at[dstb.at[0]], sca).wait()
        pltpu.make_async_copy(gb, agg_sh.at[dstb.at[1]], scb).wait()

    pltpu.sync_copy(h_hbm.at[pl.ds(r0, ROWS_PER_TILE), pl.ds(col0, HALF)],
                    h_sh.at[pl.ds(r0, ROWS_PER_TILE), :])
    pltpu.sync_copy(zeros_hbm, agg_sh.at[pl.ds(r0, ROWS_PER_TILE), :])

    loads(0, src0, dst0, e0, ldsem0)
    loads(1, src1, dst1, e1, ldsem1)
    plsc.subcore_barrier()

    def super_body(i, carry):
        wait_loads(src0, dst0, e0, ldsem0)
        process(src0, dst0, e0)

        @pl.when(i < N_SUPER // 2 - 1)
        def _():
            loads(2 * i + 2, src0, dst0, e0, ldsem0)

        wait_loads(src1, dst1, e1, ldsem1)
        process(src1, dst1, e1)

        @pl.when(i < N_SUPER // 2 - 1)
        def _():
            loads(2 * i + 3, src1, dst1, e1, ldsem1)

        return carry

    lax.fori_loop(0, N_SUPER // 2, super_body, 0)
    plsc.subcore_barrier()

    pltpu.sync_copy(agg_sh.at[pl.ds(r0, ROWS_PER_TILE), :],
                    out_hbm.at[pl.ds(r0, ROWS_PER_TILE), pl.ds(col0, HALF)])


def _gin_edge(h, e_emb, src2d, dst2d, zeros, interpret=False):
    mesh = plsc.VectorSubcoreMesh(core_axis_name="c", subcore_axis_name="s",
                                  num_cores=SC_CORES,
                                  num_subcores=SC_SUBCORES)
    idx_t = pltpu.VMEM((KSUB, SUB), jnp.int32)
    e_t = pltpu.VMEM((SUPER, HALF), jnp.float32)
    g_t = pltpu.VMEM((SUB, HALF), jnp.float32)
    sem = pltpu.SemaphoreType.DMA
    f = pl.kernel(
        _gin_edge_body,
        out_type=jax.ShapeDtypeStruct((N_NODES, EMB), jnp.float32),
        mesh=mesh,
        scratch_types=[
            pltpu.VMEM_SHARED((N_NODES, HALF), jnp.float32),
            pltpu.VMEM_SHARED((N_NODES, HALF), jnp.float32),
            idx_t, idx_t, e_t,
            idx_t, idx_t, e_t,
            g_t, g_t,
            sem, sem, sem, sem, sem, sem,
        ],
        compiler_params=pltpu.CompilerParams(use_tc_tiling_on_sc=False),
        interpret=interpret,
    )
    return f(h, e_emb, src2d, dst2d, zeros)


def kernel(x, edge_index, edge_attr, batch_vec, W_in, W_edge, W1, b1, W2, b2,
           eps, W_pred, b_pred):
    src = edge_index[0].astype(jnp.int32).reshape(IDX_ROWS, SUB)
    dst = edge_index[1].astype(jnp.int32).reshape(IDX_ROWS, SUB)
    zeros = jnp.zeros((ROWS_PER_TILE, HALF), jnp.float32)

    h = _encode(x, W_in)
    e_embs = _edge_emb_all(edge_attr, W_edge)
    for l in range(N_LAYERS):
        agg = _gin_edge(h, e_embs[l], src, dst, zeros)
        h = _mlp(h, agg, eps[l], W1[l], b1[l], W2[l], b2[l],
                 relu_out=(l < N_LAYERS - 1))
    pred, h_graph = _pool_head(h, batch_vec.astype(jnp.int32), W_pred, b_pred)
    return (pred, h_graph)


# speedup vs baseline: 4.4851x; 1.7782x over previous
import functools

import jax
import jax.numpy as jnp
from jax import lax
from jax.experimental import pallas as pl
from jax.experimental.pallas import tpu as pltpu
from jax.experimental.pallas import tpu_sc as plsc

N_NODES = 10000
N_EDGES = 320000
IN_DIM = 128
EMB = 128
EDGE_DIM = 16
N_LAYERS = 3
N_GRAPHS = 64
N_CLASS = 10

SC_CORES = 2
SC_SUBCORES = 16
HALF = EMB // SC_CORES
ROWS_PER_TILE = N_NODES // SC_SUBCORES
EDGES_PER_TILE = N_EDGES // SC_SUBCORES
SUB = 100
KSUB = 2
SUPER = SUB * KSUB
N_SUPER = EDGES_PER_TILE // SUPER
IDX_ROWS = N_EDGES // SUB


def _encode_body(x_ref, w_ref, o_ref):
    o_ref[...] = jnp.dot(x_ref[...], w_ref[...],
                         preferred_element_type=jnp.float32)


def _encode(x, w_in, interpret=False):
    bm = 2000
    return pl.pallas_call(
        _encode_body,
        grid=(N_NODES // bm,),
        in_specs=[
            pl.BlockSpec((bm, IN_DIM), lambda i: (i, 0)),
            pl.BlockSpec((IN_DIM, EMB), lambda i: (0, 0)),
        ],
        out_specs=pl.BlockSpec((bm, EMB), lambda i: (i, 0)),
        out_shape=jax.ShapeDtypeStruct((N_NODES, EMB), jnp.float32),
        interpret=interpret,
    )(x, w_in)


def _edge_emb_body(ea_ref, w_ref, o0_ref, o1_ref, o2_ref):
    ea = ea_ref[...]
    o0_ref[...] = jnp.dot(ea, w_ref[0], preferred_element_type=jnp.float32)
    o1_ref[...] = jnp.dot(ea, w_ref[1], preferred_element_type=jnp.float32)
    o2_ref[...] = jnp.dot(ea, w_ref[2], preferred_element_type=jnp.float32)


def _edge_emb_all(edge_attr, w_edge, interpret=False):
    bm = 4000
    out = jax.ShapeDtypeStruct((N_EDGES, EMB), jnp.float32)
    return pl.pallas_call(
        _edge_emb_body,
        grid=(N_EDGES // bm,),
        in_specs=[
            pl.BlockSpec((bm, EDGE_DIM), lambda i: (i, 0)),
            pl.BlockSpec((N_LAYERS, EDGE_DIM, EMB), lambda i: (0, 0, 0)),
        ],
        out_specs=[pl.BlockSpec((bm, EMB), lambda i: (i, 0))] * 3,
        out_shape=[out, out, out],
        interpret=interpret,
    )(edge_attr, w_edge)


def _mlp_body(h_ref, agg_ref, eps_ref, w1_ref, b1_ref, w2_ref, b2_ref, o_ref,
              *, relu_out):
    z = (1.0 + eps_ref[0, 0]) * h_ref[...] + agg_ref[...]
    a = jnp.maximum(jnp.dot(z, w1_ref[...],
                            preferred_element_type=jnp.float32)
                    + b1_ref[...], 0.0)
    o = jnp.dot(a, w2_ref[...], preferred_element_type=jnp.float32) + b2_ref[...]
    if relu_out:
        o = jnp.maximum(o, 0.0)
    o_ref[...] = o


def _mlp(h, agg, eps_l, w1, b1, w2, b2, relu_out, interpret=False):
    bm = 2000
    return pl.pallas_call(
        functools.partial(_mlp_body, relu_out=relu_out),
        grid=(N_NODES // bm,),
        in_specs=[
            pl.BlockSpec((bm, EMB), lambda i: (i, 0)),
            pl.BlockSpec((bm, EMB), lambda i: (i, 0)),
            pl.BlockSpec((1, 1), lambda i: (0, 0)),
            pl.BlockSpec((EMB, 2 * EMB), lambda i: (0, 0)),
            pl.BlockSpec((1, 2 * EMB), lambda i: (0, 0)),
            pl.BlockSpec((2 * EMB, EMB), lambda i: (0, 0)),
            pl.BlockSpec((1, EMB), lambda i: (0, 0)),
        ],
        out_specs=pl.BlockSpec((bm, EMB), lambda i: (i, 0)),
        out_shape=jax.ShapeDtypeStruct((N_NODES, EMB), jnp.float32),
        interpret=interpret,
    )(h, agg, eps_l.reshape(1, 1), w1, b1.reshape(1, -1), w2,
      b2.reshape(1, -1))


def _pool_head_body(h_ref, bv_ref, wp_ref, bp_ref, pred_ref, hg_ref):
    bv = bv_ref[...]
    gids = lax.broadcasted_iota(jnp.int32, (N_GRAPHS, N_NODES), 0)
    oh = (gids == bv).astype(jnp.float32)
    sums = jnp.dot(oh, h_ref[...], preferred_element_type=jnp.float32)
    counts = jnp.sum(oh, axis=1, keepdims=True)
    hg = sums / jnp.maximum(counts, 1.0)
    hg_ref[...] = hg
    pred_ref[...] = jnp.dot(hg, wp_ref[...],
                            preferred_element_type=jnp.float32) + bp_ref[...]


def _pool_head(h, batch_vec, w_pred, b_pred, interpret=False):
    return pl.pallas_call(
        _pool_head_body,
        in_specs=[
            pl.BlockSpec((N_NODES, EMB), lambda: (0, 0)),
            pl.BlockSpec((1, N_NODES), lambda: (0, 0)),
            pl.BlockSpec((EMB, N_CLASS), lambda: (0, 0)),
            pl.BlockSpec((1, N_CLASS), lambda: (0, 0)),
        ],
        out_specs=[
            pl.BlockSpec((N_GRAPHS, N_CLASS), lambda: (0, 0)),
            pl.BlockSpec((N_GRAPHS, EMB), lambda: (0, 0)),
        ],
        out_shape=[
            jax.ShapeDtypeStruct((N_GRAPHS, N_CLASS), jnp.float32),
            jax.ShapeDtypeStruct((N_GRAPHS, EMB), jnp.float32),
        ],
        interpret=interpret,
    )(h, batch_vec.reshape(1, -1), w_pred, b_pred.reshape(1, -1))


def _gin_edge_body(h_hbm, e_hbm, src_hbm, dst_hbm, zeros_hbm, out_hbm,
                   h_sh, agg_sh,
                   src0, dst0, e0, src1, dst1, e1, ga, gb,
                   ldsem0, ldsem1, gsa, gsb, sca, scb):
    cid = lax.axis_index("c")
    sid = lax.axis_index("s")
    col0 = cid * HALF
    r0 = sid * ROWS_PER_TILE
    row_base = sid * (EDGES_PER_TILE // SUB)
    e_base = sid * EDGES_PER_TILE

    def loads(c, srcb, dstb, eb, sem):
        rb = row_base + c * KSUB
        eo = e_base + c * SUPER
        pltpu.async_copy(src_hbm.at[pl.ds(rb, KSUB), :], srcb, sem)
        pltpu.async_copy(dst_hbm.at[pl.ds(rb, KSUB), :], dstb, sem)
        pltpu.async_copy(e_hbm.at[pl.ds(eo, SUPER), pl.ds(col0, HALF)], eb,
                         sem)

    def wait_loads(srcb, dstb, eb, sem):
        pltpu.make_async_copy(src_hbm.at[pl.ds(row_base, KSUB), :], srcb,
                              sem).wait()
        pltpu.make_async_copy(dst_hbm.at[pl.ds(row_base, KSUB), :], dstb,
                              sem).wait()
        pltpu.make_async_copy(
            e_hbm.at[pl.ds(e_base, SUPER), pl.ds(col0, HALF)], eb, sem).wait()

    def compute(eb, k, g):
        @plsc.parallel_loop(0, SUB, step=1, unroll=4)
        def _(j):
            row = k * SUB + j
            for v in range(HALF // 16):
                sl = pl.ds(v * 16, 16)
                g[j, sl] = jnp.maximum(g[j, sl] + eb[row, sl], 0.0)

    def process(srcb, dstb, eb):
        pltpu.async_copy(h_sh.at[srcb.at[0]], ga, gsa)
        pltpu.async_copy(h_sh.at[srcb.at[1]], gb, gsb)
        pltpu.make_async_copy(h_sh.at[srcb.at[0]], ga, gsa).wait()
        compute(eb, 0, ga)
        pltpu.async_copy(ga, agg_sh.at[dstb.at[0]], sca, add=True)
        pltpu.make_async_copy(h_sh.at[srcb.at[1]], gb, gsb).wait()
        compute(eb, 1, gb)
        pltpu.async_copy(gb, agg_sh.at[dstb.at[1]], scb, add=True)
        pltpu.make_async_copy(ga, agg_sh.at[dstb.at[0]], sca).wait()
        pltpu.make_async_copy(gb, agg_sh.at[dstb.at[1]], scb).wait()

    pltpu.sync_copy(h_hbm.at[pl.ds(r0, ROWS_PER_TILE), pl.ds(col0, HALF)],
                    h_sh.at[pl.ds(r0, ROWS_PER_TILE), :])
    pltpu.sync_copy(zeros_hbm, agg_sh.at[pl.ds(r0, ROWS_PER_TILE), :])

    loads(0, src0, dst0, e0, ldsem0)
    loads(1, src1, dst1, e1, ldsem1)
    plsc.subcore_barrier()

    def super_body(i, carry):
        wait_loads(src0, dst0, e0, ldsem0)
        process(src0, dst0, e0)

        @pl.when(i < N_SUPER // 2 - 1)
        def _():
            loads(2 * i + 2, src0, dst0, e0, ldsem0)

        wait_loads(src1, dst1, e1, ldsem1)
        process(src1, dst1, e1)

        @pl.when(i < N_SUPER // 2 - 1)
        def _():
            loads(2 * i + 3, src1, dst1, e1, ldsem1)

        return carry

    lax.fori_loop(0, N_SUPER // 2, super_body, 0)
    plsc.subcore_barrier()

    pltpu.sync_copy(agg_sh.at[pl.ds(r0, ROWS_PER_TILE), :],
                    out_hbm.at[pl.ds(r0, ROWS_PER_TILE), pl.ds(col0, HALF)])


def _gin_edge(h, e_emb, src2d, dst2d, zeros, interpret=False):
    mesh = plsc.VectorSubcoreMesh(core_axis_name="c", subcore_axis_name="s",
                                  num_cores=SC_CORES,
                                  num_subcores=SC_SUBCORES)
    idx_t = pltpu.VMEM((KSUB, SUB), jnp.int32)
    e_t = pltpu.VMEM((SUPER, HALF), jnp.float32)
    g_t = pltpu.VMEM((SUB, HALF), jnp.float32)
    sem = pltpu.SemaphoreType.DMA
    f = pl.kernel(
        _gin_edge_body,
        out_type=jax.ShapeDtypeStruct((N_NODES, EMB), jnp.float32),
        mesh=mesh,
        scratch_types=[
            pltpu.VMEM_SHARED((N_NODES, HALF), jnp.float32),
            pltpu.VMEM_SHARED((N_NODES, HALF), jnp.float32),
            idx_t, idx_t, e_t,
            idx_t, idx_t, e_t,
            g_t, g_t,
            sem, sem, sem, sem, sem, sem,
        ],
        compiler_params=pltpu.CompilerParams(use_tc_tiling_on_sc=False),
        interpret=interpret,
    )
    return f(h, e_emb, src2d, dst2d, zeros)


def kernel(x, edge_index, edge_attr, batch_vec, W_in, W_edge, W1, b1, W2, b2,
           eps, W_pred, b_pred):
    src = edge_index[0].astype(jnp.int32).reshape(IDX_ROWS, SUB)
    dst = edge_index[1].astype(jnp.int32).reshape(IDX_ROWS, SUB)
    zeros = jnp.zeros((ROWS_PER_TILE, HALF), jnp.float32)

    h = _encode(x, W_in)
    e_embs = _edge_emb_all(edge_attr, W_edge)
    for l in range(N_LAYERS):
        agg = _gin_edge(h, e_embs[l], src, dst, zeros)
        h = _mlp(h, agg, eps[l], W1[l], b1[l], W2[l], b2[l],
                 relu_out=(l < N_LAYERS - 1))
    pred, h_graph = _pool_head(h, batch_vec.astype(jnp.int32), W_pred, b_pred)
    return (pred, h_graph)
